# baseline (speedup 1.0000x reference)
import functools

import jax
import jax.numpy as jnp
from jax import lax
from jax.experimental import pallas as pl
from jax.experimental.pallas import tpu as pltpu
from jax.experimental.pallas import tpu_sc as plsc

N = 10000
D = 128
E = 320000
HID = 256

NC = 2
NS = 16
NW = NC * NS
EDGES_PER_TILE = E // NW
CH = 128
NCH = EDGES_PER_TILE // CH
TAIL = EDGES_PER_TILE - NCH * CH
PAD_N = 10240
ROWS_PER_TILE = PAD_N // NS


def _sc_aggregate(h, edge_index, zeros):
    mesh = plsc.VectorSubcoreMesh(core_axis_name="c", subcore_axis_name="s")

    @functools.partial(
        pl.kernel,
        mesh=mesh,
        out_type=jax.ShapeDtypeStruct((NC, PAD_N, D), jnp.float32),
        scratch_types=[
            pltpu.VMEM((2, CH), jnp.int32),
            pltpu.VMEM((2, CH), jnp.int32),
            pltpu.VMEM((CH, D), jnp.float32),
            pltpu.VMEM((CH, D), jnp.float32),
            pltpu.VMEM((2, TAIL), jnp.int32),
            pltpu.VMEM((TAIL, D), jnp.float32),
            pltpu.VMEM_SHARED((PAD_N, D), jnp.float32),
            pltpu.SemaphoreType.DMA,
            pltpu.SemaphoreType.DMA,
            pltpu.SemaphoreType.DMA,
            pltpu.SemaphoreType.DMA,
            pltpu.SemaphoreType.DMA,
        ],
    )
    def agg_kernel(h_hbm, edge_hbm, z_hbm, out_hbm,
                   idx_a, idx_b, rows_a, rows_b,
                   idx_t, rows_t, acc_sh,
                   sem_a, sem_b, sem_ia, sem_ib, sem_t):
        cid = lax.axis_index("c")
        sid = lax.axis_index("s")
        wid = cid * NS + sid
        row0 = sid * ROWS_PER_TILE
        tail0 = NCH * CH

        def idx_start(j, buf, sem):
            pltpu.async_copy(edge_hbm.at[:, wid, pl.ds(j * CH, CH)], buf, sem)

        def idx_wait(j, buf, sem):
            pltpu.make_async_copy(edge_hbm.at[:, wid, pl.ds(j * CH, CH)], buf, sem).wait()

        def gather_start(buf, rbuf, sem):
            pltpu.async_copy(h_hbm.at[buf.at[0]], rbuf, sem)

        def gather_wait(buf, rbuf, sem):
            pltpu.make_async_copy(h_hbm.at[buf.at[0]], rbuf, sem).wait()

        def scatter(rbuf, buf):
            pltpu.sync_copy(rbuf, acc_sh.at[buf.at[1]], add=True)

        idx_start(0, idx_a, sem_ia)
        pltpu.async_copy(edge_hbm.at[:, wid, pl.ds(tail0, TAIL)], idx_t, sem_t)
        idx_wait(0, idx_a, sem_ia)
        gather_start(idx_a, rows_a, sem_a)
        idx_start(1, idx_b, sem_ib)
        pltpu.make_async_copy(edge_hbm.at[:, wid, pl.ds(tail0, TAIL)], idx_t, sem_t).wait()
        pltpu.async_copy(h_hbm.at[idx_t.at[0]], rows_t, sem_t)
        pltpu.sync_copy(z_hbm.at[pl.ds(row0, ROWS_PER_TILE)],
                        acc_sh.at[pl.ds(row0, ROWS_PER_TILE)])
        plsc.subcore_barrier()

        @pl.loop(0, NCH - 2, step=2)
        def _(j):
            gather_wait(idx_a, rows_a, sem_a)
            idx_wait(j + 1, idx_b, sem_ib)
            gather_start(idx_b, rows_b, sem_b)
            scatter(rows_a, idx_a)
            idx_start(j + 2, idx_a, sem_ia)
            gather_wait(idx_b, rows_b, sem_b)
            idx_wait(j + 2, idx_a, sem_ia)
            gather_start(idx_a, rows_a, sem_a)
            scatter(rows_b, idx_b)
            idx_start(j + 3, idx_b, sem_ib)

        gather_wait(idx_a, rows_a, sem_a)
        idx_wait(NCH - 1, idx_b, sem_ib)
        gather_start(idx_b, rows_b, sem_b)
        scatter(rows_a, idx_a)
        gather_wait(idx_b, rows_b, sem_b)
        scatter(rows_b, idx_b)

        pltpu.make_async_copy(h_hbm.at[idx_t.at[0]], rows_t, sem_t).wait()
        pltpu.sync_copy(rows_t, acc_sh.at[idx_t.at[1]], add=True)

        plsc.subcore_barrier()

        pltpu.sync_copy(acc_sh.at[pl.ds(row0, ROWS_PER_TILE)],
                        out_hbm.at[cid, pl.ds(row0, ROWS_PER_TILE)])

    return agg_kernel(h, edge_index.reshape(2, NW, EDGES_PER_TILE), zeros)


def _tc_mlp(h, agg0, agg1, W1, b1, W2, b2, act):
    BN = 1000

    def mlp_kernel(h_ref, a0_ref, a1_ref, W1_ref, b1_ref, W2_ref, b2_ref, o_ref):
        z = h_ref[...] + a0_ref[...] + a1_ref[...]
        t = jnp.dot(z.astype(jnp.bfloat16), W1_ref[...].astype(jnp.bfloat16),
                    preferred_element_type=jnp.float32)
        t = t + b1_ref[...]
        t = jnp.where(t > 0, t, 0.2 * t)
        o = jnp.dot(t.astype(jnp.bfloat16), W2_ref[...].astype(jnp.bfloat16),
                    preferred_element_type=jnp.float32)
        o = o + b2_ref[...]
        if act:
            o = jnp.where(o > 0, o, 0.2 * o)
        o_ref[...] = o

    return pl.pallas_call(
        mlp_kernel,
        grid=(N // BN,),
        in_specs=[
            pl.BlockSpec((BN, D), lambda i: (i, 0)),
            pl.BlockSpec((BN, D), lambda i: (i, 0)),
            pl.BlockSpec((BN, D), lambda i: (i, 0)),
            pl.BlockSpec((D, HID), lambda i: (0, 0)),
            pl.BlockSpec((1, HID), lambda i: (0, 0)),
            pl.BlockSpec((HID, D), lambda i: (0, 0)),
            pl.BlockSpec((1, D), lambda i: (0, 0)),
        ],
        out_specs=pl.BlockSpec((BN, D), lambda i: (i, 0)),
        out_shape=jax.ShapeDtypeStruct((N, D), jnp.float32),
    )(h, agg0, agg1, W1, b1.reshape(1, HID), W2, b2.reshape(1, D))


def kernel(x, edge_index,
           W1_0, b1_0, W2_0, b2_0,
           W1_1, b1_1, W2_1, b2_1,
           W1_2, b1_2, W2_2, b2_2):
    zeros = jnp.zeros((PAD_N, D), jnp.float32)
    params = [(W1_0, b1_0, W2_0, b2_0),
              (W1_1, b1_1, W2_1, b2_1),
              (W1_2, b1_2, W2_2, b2_2)]
    h = x
    for l in range(3):
        agg = _sc_aggregate(h, edge_index, zeros)
        h = _tc_mlp(h, agg[0], agg[1], *params[l], act=(l < 2))
    return h


# speedup vs baseline: 1.0019x; 1.0019x over previous
import functools

import jax
import jax.numpy as jnp
from jax import lax
from jax.experimental import pallas as pl
from jax.experimental.pallas import tpu as pltpu
from jax.experimental.pallas import tpu_sc as plsc

N = 10000
D = 128
E = 320000
HID = 256

NC = 2
NS = 16
NW = NC * NS
EDGES_PER_TILE = E // NW
CH = 128
NCH = EDGES_PER_TILE // CH
TAIL = EDGES_PER_TILE - NCH * CH
PAD_N = 10240
ROWS_PER_TILE = PAD_N // NS


def _sc_aggregate(h, edge_index, zeros):
    mesh = plsc.VectorSubcoreMesh(core_axis_name="c", subcore_axis_name="s")

    @functools.partial(
        pl.kernel,
        mesh=mesh,
        out_type=jax.ShapeDtypeStruct((NC, PAD_N, D), jnp.float32),
        scratch_types=[
            pltpu.VMEM((2, CH), jnp.int32),
            pltpu.VMEM((2, CH), jnp.int32),
            pltpu.VMEM((CH, D), jnp.float32),
            pltpu.VMEM((CH, D), jnp.float32),
            pltpu.VMEM((2, TAIL), jnp.int32),
            pltpu.VMEM((TAIL, D), jnp.float32),
            pltpu.VMEM_SHARED((PAD_N, D), jnp.float32),
            pltpu.SemaphoreType.DMA,
            pltpu.SemaphoreType.DMA,
            pltpu.SemaphoreType.DMA,
            pltpu.SemaphoreType.DMA,
            pltpu.SemaphoreType.DMA,
        ],
    )
    def agg_kernel(h_hbm, edge_hbm, z_hbm, out_hbm,
                   idx_a, idx_b, rows_a, rows_b,
                   idx_t, rows_t, acc_sh,
                   sem_a, sem_b, sem_ia, sem_ib, sem_t):
        cid = lax.axis_index("c")
        sid = lax.axis_index("s")
        wid = cid * NS + sid
        row0 = sid * ROWS_PER_TILE
        tail0 = NCH * CH

        def idx_start(j, buf, sem):
            pltpu.async_copy(edge_hbm.at[:, wid, pl.ds(j * CH, CH)], buf, sem)

        def idx_wait(j, buf, sem):
            pltpu.make_async_copy(edge_hbm.at[:, wid, pl.ds(j * CH, CH)], buf, sem).wait()

        def gather_start(buf, rbuf, sem):
            pltpu.async_copy(h_hbm.at[buf.at[0]], rbuf, sem)

        def gather_wait(buf, rbuf, sem):
            pltpu.make_async_copy(h_hbm.at[buf.at[0]], rbuf, sem).wait()

        def scatter(rbuf, buf):
            pltpu.sync_copy(rbuf, acc_sh.at[buf.at[1]], add=True)

        idx_start(0, idx_a, sem_ia)
        pltpu.async_copy(edge_hbm.at[:, wid, pl.ds(tail0, TAIL)], idx_t, sem_t)
        idx_wait(0, idx_a, sem_ia)
        gather_start(idx_a, rows_a, sem_a)
        idx_start(1, idx_b, sem_ib)
        pltpu.make_async_copy(edge_hbm.at[:, wid, pl.ds(tail0, TAIL)], idx_t, sem_t).wait()
        pltpu.async_copy(h_hbm.at[idx_t.at[0]], rows_t, sem_t)
        pltpu.sync_copy(z_hbm.at[pl.ds(row0, ROWS_PER_TILE)],
                        acc_sh.at[pl.ds(row0, ROWS_PER_TILE)])
        plsc.subcore_barrier()

        @pl.loop(0, NCH - 2, step=2)
        def _(j):
            gather_wait(idx_a, rows_a, sem_a)
            idx_wait(j + 1, idx_b, sem_ib)
            gather_start(idx_b, rows_b, sem_b)
            scatter(rows_a, idx_a)
            idx_start(j + 2, idx_a, sem_ia)
            gather_wait(idx_b, rows_b, sem_b)
            idx_wait(j + 2, idx_a, sem_ia)
            gather_start(idx_a, rows_a, sem_a)
            scatter(rows_b, idx_b)
            idx_start(j + 3, idx_b, sem_ib)

        gather_wait(idx_a, rows_a, sem_a)
        idx_wait(NCH - 1, idx_b, sem_ib)
        gather_start(idx_b, rows_b, sem_b)
        scatter(rows_a, idx_a)
        gather_wait(idx_b, rows_b, sem_b)
        scatter(rows_b, idx_b)

        pltpu.make_async_copy(h_hbm.at[idx_t.at[0]], rows_t, sem_t).wait()
        pltpu.sync_copy(rows_t, acc_sh.at[idx_t.at[1]], add=True)

        plsc.subcore_barrier()

        pltpu.sync_copy(acc_sh.at[pl.ds(row0, ROWS_PER_TILE)],
                        out_hbm.at[cid, pl.ds(row0, ROWS_PER_TILE)])

    return agg_kernel(h, edge_index.reshape(2, NW, EDGES_PER_TILE), zeros)


def _tc_mlp(h, agg0, agg1, W1, b1, W2, b2, act):
    BN = 1000

    def mlp_kernel(h_ref, a0_ref, a1_ref, W1_ref, b1_ref, W2_ref, b2_ref, o_ref):
        z = h_ref[...] + a0_ref[...] + a1_ref[...]
        t = jnp.dot(z, W1_ref[...], preferred_element_type=jnp.float32)
        t = t + b1_ref[...]
        t = jnp.where(t > 0, t, 0.2 * t)
        o = jnp.dot(t, W2_ref[...], preferred_element_type=jnp.float32)
        o = o + b2_ref[...]
        if act:
            o = jnp.where(o > 0, o, 0.2 * o)
        o_ref[...] = o

    return pl.pallas_call(
        mlp_kernel,
        grid=(N // BN,),
        in_specs=[
            pl.BlockSpec((BN, D), lambda i: (i, 0)),
            pl.BlockSpec((BN, D), lambda i: (i, 0)),
            pl.BlockSpec((BN, D), lambda i: (i, 0)),
            pl.BlockSpec((D, HID), lambda i: (0, 0)),
            pl.BlockSpec((1, HID), lambda i: (0, 0)),
            pl.BlockSpec((HID, D), lambda i: (0, 0)),
            pl.BlockSpec((1, D), lambda i: (0, 0)),
        ],
        out_specs=pl.BlockSpec((BN, D), lambda i: (i, 0)),
        out_shape=jax.ShapeDtypeStruct((N, D), jnp.float32),
    )(h, agg0, agg1, W1, b1.reshape(1, HID), W2, b2.reshape(1, D))


def kernel(x, edge_index,
           W1_0, b1_0, W2_0, b2_0,
           W1_1, b1_1, W2_1, b2_1,
           W1_2, b1_2, W2_2, b2_2):
    zeros = jnp.zeros((PAD_N, D), jnp.float32)
    params = [(W1_0, b1_0, W2_0, b2_0),
              (W1_1, b1_1, W2_1, b2_1),
              (W1_2, b1_2, W2_2, b2_2)]
    h = x
    for l in range(3):
        agg = _sc_aggregate(h, edge_index, zeros)
        h = _tc_mlp(h, agg[0], agg[1], *params[l], act=(l < 2))
    return h
